# baseline (speedup 1.0000x reference)
import jax
import jax.numpy as jnp
from jax import lax
from jax.experimental import pallas as pl
from jax.experimental.pallas import tpu as pltpu

ROWS = 128
COLS = 100000
RB = 8
TW = 2048
NT = (COLS + TW - 1) // TW
NR = ROWS // RB

_R0 = (13, 15, 26, 6)
_R1 = (17, 29, 16, 24)
_KS0 = 0
_KS1 = 42
_KS2 = _KS0 ^ _KS1 ^ 0x1BD11BDA


def _round_group(x0, x1, rots):
    for r in rots:
        x0 = x0 + x1
        x1 = ((x1 << jnp.uint32(r)) | (x1 >> jnp.uint32(32 - r))) ^ x0
    return x0, x1


def _threefry_bits(n):
    ks0 = jnp.uint32(_KS0)
    ks1 = jnp.uint32(_KS1)
    ks2 = jnp.uint32(_KS2)
    x0 = jnp.zeros_like(n)
    x1 = n + ks1
    x0, x1 = _round_group(x0, x1, _R0)
    x0 = x0 + ks1
    x1 = x1 + jnp.uint32(_KS2 + 1)
    x0, x1 = _round_group(x0, x1, _R1)
    x0 = x0 + ks2
    x1 = x1 + jnp.uint32(_KS0 + 2)
    x0, x1 = _round_group(x0, x1, _R0)
    x0 = x0 + ks0
    x1 = x1 + jnp.uint32(_KS1 + 3)
    x0, x1 = _round_group(x0, x1, _R1)
    x0 = x0 + ks1
    x1 = x1 + jnp.uint32(_KS2 + 4)
    x0, x1 = _round_group(x0, x1, _R0)
    x0 = x0 + ks2
    x1 = x1 + jnp.uint32(_KS0 + 5)
    return x0 ^ x1


def _kernel(logits_ref, out_ref, e_scr, s_scr, r_scr):
    rb = pl.program_id(0)
    ph = pl.program_id(1)
    c = pl.program_id(2)

    @pl.when(ph == 0)
    def _phase0():
        @pl.when(c == 0)
        def _init():
            s_scr[...] = jnp.zeros((RB, TW), jnp.float32)

        row = rb * RB + lax.broadcasted_iota(jnp.int32, (RB, TW), 0)
        col = c * TW + lax.broadcasted_iota(jnp.int32, (RB, TW), 1)
        n = (row * COLS + col).astype(jnp.uint32)
        bits = _threefry_bits(n)
        fb = (bits >> jnp.uint32(9)) | jnp.uint32(0x3F800000)
        u = lax.bitcast_convert_type(fb, jnp.float32) - jnp.float32(1.0)
        eps = jnp.float32(1e-10)
        g = -jnp.log(eps - jnp.log(u + eps))
        z = logits_ref[...] + g
        e = jnp.exp(z)
        e = jnp.where(col < COLS, e, jnp.float32(0.0))
        e_scr[c] = e
        s_scr[...] += e

    @pl.when(ph == 1)
    def _phase1():
        @pl.when(c == 0)
        def _recip():
            s = jnp.sum(s_scr[...], axis=1, keepdims=True)
            r_scr[...] = jnp.broadcast_to(jnp.float32(1.0) / s, (RB, TW))

        out_ref[...] = e_scr[c] * r_scr[...]


def kernel(logits):
    return pl.pallas_call(
        _kernel,
        grid=(NR, 2, NT),
        in_specs=[
            pl.BlockSpec((RB, TW),
                         lambda r, ph, c: (r, jnp.where(ph == 0, c, 0))),
        ],
        out_specs=pl.BlockSpec((RB, TW),
                               lambda r, ph, c: (r, jnp.where(ph == 1, c, 0))),
        out_shape=jax.ShapeDtypeStruct((ROWS, COLS), jnp.float32),
        scratch_shapes=[
            pltpu.VMEM((NT, RB, TW), jnp.float32),
            pltpu.VMEM((RB, TW), jnp.float32),
            pltpu.VMEM((RB, TW), jnp.float32),
        ],
    )(logits)
